# baseline (speedup 1.0000x reference)
import functools

import jax
import jax.numpy as jnp
from jax.experimental import pallas as pl
from jax.experimental.pallas import tpu as pltpu

K_TOP = 8


def _tc_body(x_ref, out_ref):
    S = x_ref.shape[1]
    C = x_ref.shape[2]
    nstep = S // 8
    neg = jnp.full((8, C), -jnp.inf, dtype=jnp.float32)

    def step(i, T):
        d = x_ref[0, pl.ds(i * 8, 8), :]
        out = []
        for k in range(K_TOP):
            t = T[k]
            out.append(jnp.maximum(t, d))
            if k < K_TOP - 1:
                d = jnp.minimum(t, d)
        return tuple(out)

    T = jax.lax.fori_loop(0, nstep, step, tuple([neg] * K_TOP), unroll=4)

    cand = jnp.concatenate(T, axis=0)
    ridx = jax.lax.broadcasted_iota(jnp.int32, (8 * K_TOP, C), 0)
    outs = []
    for _ in range(K_TOP):
        m = jnp.max(cand, axis=0)
        eq = cand == m[None, :]
        first = jnp.min(jnp.where(eq, ridx, 8 * K_TOP), axis=0)
        cand = jnp.where(eq & (ridx == first[None, :]), -jnp.inf, cand)
        outs.append(m)
    res = jnp.stack(outs, axis=0)
    out_ref[0] = jnp.transpose(res, (1, 0))


def _kmax_tc(x, b_lo=0, b_hi=None):
    B, S, C = x.shape
    if b_hi is None:
        b_hi = B
    nb = b_hi - b_lo
    out = pl.pallas_call(
        _tc_body,
        grid=(nb,),
        in_specs=[pl.BlockSpec((1, S, C), lambda b: (b + b_lo, 0, 0))],
        out_specs=pl.BlockSpec((1, C, K_TOP), lambda b: (b, 0, 0)),
        out_shape=jax.ShapeDtypeStruct((nb, C, K_TOP), jnp.float32),
    )(x)
    return out.reshape(nb, C * K_TOP)


_SORT8 = [
    (0, 1), (2, 3), (4, 5), (6, 7),
    (0, 2), (1, 3), (4, 6), (5, 7),
    (1, 2), (5, 6),
    (0, 4), (1, 5), (2, 6), (3, 7),
    (2, 4), (3, 5),
    (1, 2), (3, 4), (5, 6),
]


def _insert(T, v):
    out = []
    d = v
    for kk in range(K_TOP):
        tk = T[kk]
        out.append(jnp.maximum(tk, d))
        if kk < K_TOP - 1:
            d = jnp.minimum(tk, d)
    return tuple(out)


def _kmax_sc(x, b_lo=0, b_hi=None, chunk=256):
    from jax.experimental.pallas import tpu_sc as plsc

    B, S, C = x.shape
    if b_hi is None:
        b_hi = B
    nb = b_hi - b_lo
    G = C // 16
    NW = 32
    assert NW % nb == 0
    nsh = NW // nb
    rows = S // nsh
    nchunks = rows // chunk
    assert rows % chunk == 0 and chunk % 8 == 0

    mesh = plsc.VectorSubcoreMesh(core_axis_name="c", subcore_axis_name="s")

    @functools.partial(
        pl.kernel,
        out_type=jax.ShapeDtypeStruct((nb, nsh * G, 8 * 16), jnp.float32),
        mesh=mesh,
        scratch_types=[
            pltpu.VMEM((chunk, C), jnp.float32),
            pltpu.VMEM((G * K_TOP, 16), jnp.float32),
            pltpu.VMEM((G, 8 * 16), jnp.float32),
        ],
        compiler_params=pltpu.CompilerParams(needs_layout_passes=False),
    )
    def k1(x_hbm, cand_hbm, buf, tbuf, obuf):
        wid = jax.lax.axis_index("s") * 2 + jax.lax.axis_index("c")
        b = wid // nsh
        sh = wid % nsh
        lanes8 = jax.lax.iota(jnp.int32, 16) * K_TOP
        neg = jnp.full((16,), -jnp.inf, dtype=jnp.float32)

        def initg(g, _):
            def initk(i, _):
                tbuf[g * K_TOP + i] = neg
                return 0
            return jax.lax.fori_loop(0, K_TOP, initk, 0)

        jax.lax.fori_loop(0, G, initg, 0)

        row0 = sh * rows

        def do_chunk(ci, _):
            pltpu.sync_copy(
                x_hbm.at[b + b_lo, pl.ds(row0 + ci * chunk, chunk), :], buf
            )

            def do_group(g, _):
                T = tuple(tbuf[g * K_TOP + kk] for kk in range(K_TOP))

                def do_rows(r, T):
                    base = r * 8
                    v = [buf[base + i, pl.ds(g * 16, 16)] for i in range(8)]
                    m01 = jnp.maximum(v[0], v[1])
                    m23 = jnp.maximum(v[2], v[3])
                    m45 = jnp.maximum(v[4], v[5])
                    m67 = jnp.maximum(v[6], v[7])
                    m = jnp.maximum(
                        jnp.maximum(m01, m23), jnp.maximum(m45, m67)
                    )
                    pred = jnp.any(m > T[K_TOP - 1])

                    def slow(T):
                        s = list(v)
                        for (i, j) in _SORT8:
                            hi = jnp.maximum(s[i], s[j])
                            lo = jnp.minimum(s[i], s[j])
                            s[i], s[j] = hi, lo
                        T = _insert(T, s[0])
                        for j in range(1, 8):
                            pj = jnp.any(s[j] > T[K_TOP - 1])
                            T = jax.lax.cond(
                                pj, _insert, lambda T, v: T, T, s[j]
                            )
                        return T

                    return jax.lax.cond(pred, slow, lambda T: T, T)

                T = jax.lax.fori_loop(0, chunk // 8, do_rows, T)
                for kk in range(K_TOP):
                    tbuf[g * K_TOP + kk] = T[kk]
                return 0

            jax.lax.fori_loop(0, G, do_group, 0)
            return 0

        jax.lax.fori_loop(0, nchunks, do_chunk, 0)

        def emit(g, _):
            for kk in range(K_TOP):
                plsc.store_scatter(
                    obuf, [jnp.full((16,), g, jnp.int32), lanes8 + kk],
                    tbuf[g * K_TOP + kk],
                )
            return 0

        jax.lax.fori_loop(0, G, emit, 0)
        pltpu.sync_copy(obuf, cand_hbm.at[b, pl.ds(sh * G, G), :])

    cand = k1(x)
    if nsh == 1:
        return cand.reshape(nb, C * K_TOP)

    @functools.partial(
        pl.kernel,
        out_type=jax.ShapeDtypeStruct((nb, G, 8 * 16), jnp.float32),
        mesh=mesh,
        scratch_types=[
            pltpu.VMEM((nsh * G, 8 * 16), jnp.float32),
            pltpu.VMEM((G, 8 * 16), jnp.float32),
        ],
        compiler_params=pltpu.CompilerParams(needs_layout_passes=False),
    )
    def k2(cand_hbm, out_hbm, cbuf, obuf):
        wid = jax.lax.axis_index("s") * 2 + jax.lax.axis_index("c")
        lanes8 = jax.lax.iota(jnp.int32, 16) * K_TOP
        neg = jnp.full((16,), -jnp.inf, dtype=jnp.float32)

        @pl.when(wid < nb)
        def _():
            pltpu.sync_copy(cand_hbm.at[wid], cbuf)

            def do_group(g, _):
                T = tuple([neg] * K_TOP)
                for sh in range(nsh):
                    for kk in range(K_TOP):
                        row = jnp.full((16,), sh * G + g, jnp.int32)
                        v = plsc.load_gather(cbuf, [row, lanes8 + kk])
                        T = _insert(T, v)
                for kk in range(K_TOP):
                    plsc.store_scatter(
                        obuf, [jnp.full((16,), g, jnp.int32), lanes8 + kk],
                        T[kk],
                    )
                return 0

            jax.lax.fori_loop(0, G, do_group, 0)
            pltpu.sync_copy(obuf, out_hbm.at[wid])

    return k2(cand).reshape(nb, C * K_TOP)


_SC_BATCHES = 16


def kernel(inputs):
    b_sc = _SC_BATCHES
    B = inputs.shape[0]
    out_sc = _kmax_sc(inputs, 0, b_sc)
    out_tc = _kmax_tc(inputs, b_sc, B)
    return jnp.concatenate([out_sc, out_tc], axis=0)


# speedup vs baseline: 2.5102x; 2.5102x over previous
import functools

import jax
import jax.numpy as jnp
from jax.experimental import pallas as pl
from jax.experimental.pallas import tpu as pltpu

K_TOP = 8


def _tc_body(x_ref, out_ref):
    S = x_ref.shape[1]
    C = x_ref.shape[2]
    nstep = S // 8
    neg = jnp.full((8, C), -jnp.inf, dtype=jnp.float32)

    def step(i, T):
        d = x_ref[0, pl.ds(i * 8, 8), :]
        out = []
        for k in range(K_TOP):
            t = T[k]
            out.append(jnp.maximum(t, d))
            if k < K_TOP - 1:
                d = jnp.minimum(t, d)
        return tuple(out)

    T = jax.lax.fori_loop(0, nstep, step, tuple([neg] * K_TOP), unroll=4)

    cand = jnp.concatenate(T, axis=0)
    ridx = jax.lax.broadcasted_iota(jnp.int32, (8 * K_TOP, C), 0)
    outs = []
    for _ in range(K_TOP):
        m = jnp.max(cand, axis=0)
        eq = cand == m[None, :]
        first = jnp.min(jnp.where(eq, ridx, 8 * K_TOP), axis=0)
        cand = jnp.where(eq & (ridx == first[None, :]), -jnp.inf, cand)
        outs.append(m)
    res = jnp.stack(outs, axis=0)
    out_ref[0] = jnp.transpose(res, (1, 0))


def _kmax_tc(x, b_lo=0, b_hi=None):
    B, S, C = x.shape
    if b_hi is None:
        b_hi = B
    nb = b_hi - b_lo
    out = pl.pallas_call(
        _tc_body,
        grid=(nb,),
        in_specs=[pl.BlockSpec((1, S, C), lambda b: (b + b_lo, 0, 0))],
        out_specs=pl.BlockSpec((1, C, K_TOP), lambda b: (b, 0, 0)),
        out_shape=jax.ShapeDtypeStruct((nb, C, K_TOP), jnp.float32),
    )(x)
    return out.reshape(nb, C * K_TOP)


_GRP = 16


def _insert(T, v):
    out = []
    d = v
    for kk in range(K_TOP):
        tk = T[kk]
        out.append(jnp.maximum(tk, d))
        if kk < K_TOP - 1:
            d = jnp.minimum(tk, d)
    return tuple(out)


def _kmax_sc(x, b_lo=0, b_hi=None, chunk=256):
    from jax.experimental.pallas import tpu_sc as plsc

    B, S, C = x.shape
    if b_hi is None:
        b_hi = B
    nb = b_hi - b_lo
    G = C // 16
    NW = 32
    assert NW % nb == 0
    nsh = NW // nb
    rows = S // nsh
    nchunks = rows // chunk
    assert rows % chunk == 0 and chunk % 8 == 0

    mesh = plsc.VectorSubcoreMesh(core_axis_name="c", subcore_axis_name="s")

    @functools.partial(
        pl.kernel,
        out_type=jax.ShapeDtypeStruct((nb, nsh * G, 8 * 16), jnp.float32),
        mesh=mesh,
        scratch_types=[
            pltpu.VMEM((chunk, C), jnp.float32),
            pltpu.VMEM((G * K_TOP, 16), jnp.float32),
            pltpu.VMEM((G, 8 * 16), jnp.float32),
        ],
        compiler_params=pltpu.CompilerParams(needs_layout_passes=False),
    )
    def k1(x_hbm, cand_hbm, buf, tbuf, obuf):
        wid = jax.lax.axis_index("s") * 2 + jax.lax.axis_index("c")
        b = wid // nsh
        sh = wid % nsh
        lanes8 = jax.lax.iota(jnp.int32, 16) * K_TOP
        neg = jnp.full((16,), -jnp.inf, dtype=jnp.float32)

        for g in range(G):
            for kk in range(K_TOP):
                tbuf[g * K_TOP + kk] = neg

        row0 = sh * rows

        def do_chunk(ci, _):
            pltpu.sync_copy(
                x_hbm.at[b + b_lo, pl.ds(row0 + ci * chunk, chunk), :], buf
            )

            for g in range(G):
                T = tuple(tbuf[g * K_TOP + kk] for kk in range(K_TOP))

                def do_rows(r, T, g=g):
                    base = r * _GRP
                    v = [
                        buf[base + i, g * 16:(g + 1) * 16]
                        for i in range(_GRP)
                    ]
                    m = v[0]
                    lvl = list(v)
                    while len(lvl) > 1:
                        lvl = [
                            jnp.maximum(lvl[2 * i], lvl[2 * i + 1])
                            for i in range(len(lvl) // 2)
                        ]
                    m = lvl[0]
                    cnt = plsc.all_reduce_population_count(m > T[K_TOP - 1])
                    pred = cnt[0] > 0

                    def slow(T):
                        for i in range(_GRP):
                            T = _insert(T, v[i])
                        return T

                    return jax.lax.cond(pred, slow, lambda T: T, T)

                T = jax.lax.fori_loop(0, chunk // _GRP, do_rows, T)
                for kk in range(K_TOP):
                    tbuf[g * K_TOP + kk] = T[kk]
            return 0

        jax.lax.fori_loop(0, nchunks, do_chunk, 0)

        for g in range(G):
            for kk in range(K_TOP):
                plsc.store_scatter(
                    obuf, [jnp.full((16,), g, jnp.int32), lanes8 + kk],
                    tbuf[g * K_TOP + kk],
                )
        pltpu.sync_copy(obuf, cand_hbm.at[b, pl.ds(sh * G, G), :])

    cand = k1(x)
    if nsh == 1:
        return cand.reshape(nb, C * K_TOP)

    @functools.partial(
        pl.kernel,
        out_type=jax.ShapeDtypeStruct((nb, G, 8 * 16), jnp.float32),
        mesh=mesh,
        scratch_types=[
            pltpu.VMEM((nsh * G, 8 * 16), jnp.float32),
            pltpu.VMEM((G, 8 * 16), jnp.float32),
        ],
        compiler_params=pltpu.CompilerParams(needs_layout_passes=False),
    )
    def k2(cand_hbm, out_hbm, cbuf, obuf):
        wid = jax.lax.axis_index("s") * 2 + jax.lax.axis_index("c")
        lanes8 = jax.lax.iota(jnp.int32, 16) * K_TOP
        neg = jnp.full((16,), -jnp.inf, dtype=jnp.float32)

        @pl.when(wid < nb)
        def _():
            pltpu.sync_copy(cand_hbm.at[wid], cbuf)

            def do_group(g, _):
                T = tuple([neg] * K_TOP)
                for sh in range(nsh):
                    for kk in range(K_TOP):
                        row = jnp.full((16,), sh * G + g, jnp.int32)
                        v = plsc.load_gather(cbuf, [row, lanes8 + kk])
                        T = _insert(T, v)
                for kk in range(K_TOP):
                    plsc.store_scatter(
                        obuf, [jnp.full((16,), g, jnp.int32), lanes8 + kk],
                        T[kk],
                    )
                return 0

            jax.lax.fori_loop(0, G, do_group, 0)
            pltpu.sync_copy(obuf, out_hbm.at[wid])

    return k2(cand).reshape(nb, C * K_TOP)


_SC_BATCHES = 16


def kernel(inputs):
    b_sc = _SC_BATCHES
    B = inputs.shape[0]
    out_sc = _kmax_sc(inputs, 0, b_sc)
    out_tc = _kmax_tc(inputs, b_sc, B)
    return jnp.concatenate([out_sc, out_tc], axis=0)


# speedup vs baseline: 2.5141x; 1.0016x over previous
import functools

import jax
import jax.numpy as jnp
from jax.experimental import pallas as pl
from jax.experimental.pallas import tpu as pltpu

K_TOP = 8


def _tc_body(x_ref, out_ref):
    S = x_ref.shape[1]
    C = x_ref.shape[2]
    nstep = S // 8
    neg = jnp.full((8, C), -jnp.inf, dtype=jnp.float32)

    def step(i, T):
        d = x_ref[0, pl.ds(i * 8, 8), :]
        out = []
        for k in range(K_TOP):
            t = T[k]
            out.append(jnp.maximum(t, d))
            if k < K_TOP - 1:
                d = jnp.minimum(t, d)
        return tuple(out)

    T = jax.lax.fori_loop(0, nstep, step, tuple([neg] * K_TOP), unroll=4)

    cand = jnp.concatenate(T, axis=0)
    ridx = jax.lax.broadcasted_iota(jnp.int32, (8 * K_TOP, C), 0)
    outs = []
    for _ in range(K_TOP):
        m = jnp.max(cand, axis=0)
        eq = cand == m[None, :]
        first = jnp.min(jnp.where(eq, ridx, 8 * K_TOP), axis=0)
        cand = jnp.where(eq & (ridx == first[None, :]), -jnp.inf, cand)
        outs.append(m)
    res = jnp.stack(outs, axis=0)
    out_ref[0] = jnp.transpose(res, (1, 0))


def _kmax_tc(x, b_lo=0, b_hi=None):
    B, S, C = x.shape
    if b_hi is None:
        b_hi = B
    nb = b_hi - b_lo
    out = pl.pallas_call(
        _tc_body,
        grid=(nb,),
        in_specs=[pl.BlockSpec((1, S, C), lambda b: (b + b_lo, 0, 0))],
        out_specs=pl.BlockSpec((1, C, K_TOP), lambda b: (b, 0, 0)),
        out_shape=jax.ShapeDtypeStruct((nb, C, K_TOP), jnp.float32),
    )(x)
    return out.reshape(nb, C * K_TOP)


_GRP = 16


def _insert(T, v):
    out = []
    d = v
    for kk in range(K_TOP):
        tk = T[kk]
        out.append(jnp.maximum(tk, d))
        if kk < K_TOP - 1:
            d = jnp.minimum(tk, d)
    return tuple(out)


def _kmax_sc(x, b_lo=0, b_hi=None, chunk=256):
    from jax.experimental.pallas import tpu_sc as plsc

    B, S, C = x.shape
    if b_hi is None:
        b_hi = B
    nb = b_hi - b_lo
    G = C // 16
    NW = 32
    assert NW % nb == 0
    nsh = NW // nb
    rows = S // nsh
    nchunks = rows // chunk
    assert rows % chunk == 0 and chunk % 8 == 0

    mesh = plsc.VectorSubcoreMesh(core_axis_name="c", subcore_axis_name="s")

    @functools.partial(
        pl.kernel,
        out_type=jax.ShapeDtypeStruct((nb, nsh * G, 8 * 16), jnp.float32),
        mesh=mesh,
        scratch_types=[
            pltpu.VMEM((chunk, C), jnp.float32),
            pltpu.VMEM((G * K_TOP, 16), jnp.float32),
            pltpu.VMEM((G, 8 * 16), jnp.float32),
        ],
        compiler_params=pltpu.CompilerParams(needs_layout_passes=False),
    )
    def k1(x_hbm, cand_hbm, buf, tbuf, obuf):
        wid = jax.lax.axis_index("s") * 2 + jax.lax.axis_index("c")
        b = wid // nsh
        sh = wid % nsh
        lanes8 = jax.lax.iota(jnp.int32, 16) * K_TOP
        neg = jnp.full((16,), -jnp.inf, dtype=jnp.float32)

        for g in range(G):
            for kk in range(K_TOP):
                tbuf[g * K_TOP + kk] = neg

        row0 = sh * rows

        def do_chunk(ci, _):
            pltpu.sync_copy(
                x_hbm.at[b + b_lo, pl.ds(row0 + ci * chunk, chunk), :], buf
            )

            for g in range(G):
                T = tuple(tbuf[g * K_TOP + kk] for kk in range(K_TOP))

                def do_rows(r, T, g=g):
                    base = r * _GRP
                    v = [
                        buf[base + i, g * 16:(g + 1) * 16]
                        for i in range(_GRP)
                    ]
                    m = v[0]
                    lvl = list(v)
                    while len(lvl) > 1:
                        lvl = [
                            jnp.maximum(lvl[2 * i], lvl[2 * i + 1])
                            for i in range(len(lvl) // 2)
                        ]
                    m = lvl[0]
                    cnt = plsc.all_reduce_population_count(m > T[K_TOP - 1])
                    pred = cnt[0] > 0

                    def slow(T):
                        for i in range(_GRP):
                            T = _insert(T, v[i])
                        return T

                    return jax.lax.cond(pred, slow, lambda T: T, T)

                T = jax.lax.fori_loop(0, chunk // _GRP, do_rows, T)
                for kk in range(K_TOP):
                    tbuf[g * K_TOP + kk] = T[kk]
            return 0

        jax.lax.fori_loop(0, nchunks, do_chunk, 0)

        for g in range(G):
            for kk in range(K_TOP):
                plsc.store_scatter(
                    obuf, [jnp.full((16,), g, jnp.int32), lanes8 + kk],
                    tbuf[g * K_TOP + kk],
                )
        pltpu.sync_copy(obuf, cand_hbm.at[b, pl.ds(sh * G, G), :])

    cand = k1(x)
    if nsh == 1:
        return cand.reshape(nb, C * K_TOP)

    @functools.partial(
        pl.kernel,
        out_type=jax.ShapeDtypeStruct((nb, G, 8 * 16), jnp.float32),
        mesh=mesh,
        scratch_types=[
            pltpu.VMEM((nsh * G, 8 * 16), jnp.float32),
            pltpu.VMEM((G, 8 * 16), jnp.float32),
        ],
        compiler_params=pltpu.CompilerParams(needs_layout_passes=False),
    )
    def k2(cand_hbm, out_hbm, cbuf, obuf):
        wid = jax.lax.axis_index("s") * 2 + jax.lax.axis_index("c")
        lanes8 = jax.lax.iota(jnp.int32, 16) * K_TOP
        neg = jnp.full((16,), -jnp.inf, dtype=jnp.float32)

        @pl.when(wid < nb)
        def _():
            pltpu.sync_copy(cand_hbm.at[wid], cbuf)

            def do_group(g, _):
                T = tuple([neg] * K_TOP)
                for sh in range(nsh):
                    for kk in range(K_TOP):
                        row = jnp.full((16,), sh * G + g, jnp.int32)
                        v = plsc.load_gather(cbuf, [row, lanes8 + kk])
                        T = _insert(T, v)
                for kk in range(K_TOP):
                    plsc.store_scatter(
                        obuf, [jnp.full((16,), g, jnp.int32), lanes8 + kk],
                        T[kk],
                    )
                return 0

            jax.lax.fori_loop(0, G, do_group, 0)
            pltpu.sync_copy(obuf, out_hbm.at[wid])

    return k2(cand).reshape(nb, C * K_TOP)


_SC_BATCHES = 16


def kernel(inputs):
    b_sc = _SC_BATCHES
    B = inputs.shape[0]
    out_tc = _kmax_tc(inputs, b_sc, B)
    out_sc = _kmax_sc(inputs, 0, b_sc)
    return jnp.concatenate([out_sc, out_tc], axis=0)


# speedup vs baseline: 3.1880x; 1.2680x over previous
import functools

import jax
import jax.numpy as jnp
from jax.experimental import pallas as pl
from jax.experimental.pallas import tpu as pltpu

K_TOP = 8


def _tc_body(x_ref, out_ref):
    S = x_ref.shape[1]
    C = x_ref.shape[2]
    nstep = S // 8
    neg = jnp.full((8, C), -jnp.inf, dtype=jnp.float32)

    def step(i, T):
        d = x_ref[0, pl.ds(i * 8, 8), :]
        out = []
        for k in range(K_TOP):
            t = T[k]
            out.append(jnp.maximum(t, d))
            if k < K_TOP - 1:
                d = jnp.minimum(t, d)
        return tuple(out)

    T = jax.lax.fori_loop(0, nstep, step, tuple([neg] * K_TOP), unroll=4)

    cand = jnp.concatenate(T, axis=0)
    ridx = jax.lax.broadcasted_iota(jnp.int32, (8 * K_TOP, C), 0)
    outs = []
    for _ in range(K_TOP):
        m = jnp.max(cand, axis=0)
        eq = cand == m[None, :]
        first = jnp.min(jnp.where(eq, ridx, 8 * K_TOP), axis=0)
        cand = jnp.where(eq & (ridx == first[None, :]), -jnp.inf, cand)
        outs.append(m)
    res = jnp.stack(outs, axis=0)
    out_ref[0] = jnp.transpose(res, (1, 0))


def _kmax_tc(x, b_lo=0, b_hi=None):
    B, S, C = x.shape
    if b_hi is None:
        b_hi = B
    nb = b_hi - b_lo
    out = pl.pallas_call(
        _tc_body,
        grid=(nb,),
        in_specs=[pl.BlockSpec((1, S, C), lambda b: (b + b_lo, 0, 0))],
        out_specs=pl.BlockSpec((1, C, K_TOP), lambda b: (b, 0, 0)),
        out_shape=jax.ShapeDtypeStruct((nb, C, K_TOP), jnp.float32),
    )(x)
    return out.reshape(nb, C * K_TOP)


_GRP = 16


def _insert(T, v):
    out = []
    d = v
    for kk in range(K_TOP):
        tk = T[kk]
        out.append(jnp.maximum(tk, d))
        if kk < K_TOP - 1:
            d = jnp.minimum(tk, d)
    return tuple(out)


def _kmax_sc(x, b_lo=0, b_hi=None, chunk=256):
    from jax.experimental.pallas import tpu_sc as plsc

    B, S, C = x.shape
    if b_hi is None:
        b_hi = B
    nb = b_hi - b_lo
    G = C // 16
    NW = 32
    assert NW % nb == 0
    nsh = NW // nb
    rows = S // nsh
    nchunks = rows // chunk
    assert rows % chunk == 0 and chunk % 8 == 0

    mesh = plsc.VectorSubcoreMesh(core_axis_name="c", subcore_axis_name="s")

    @functools.partial(
        pl.kernel,
        out_type=jax.ShapeDtypeStruct((nb, nsh * G, 8 * 16), jnp.float32),
        mesh=mesh,
        scratch_types=[
            pltpu.VMEM((chunk, C), jnp.float32),
            pltpu.VMEM((G * K_TOP, 16), jnp.float32),
            pltpu.VMEM((G, 8 * 16), jnp.float32),
        ],
        compiler_params=pltpu.CompilerParams(needs_layout_passes=False),
    )
    def k1(x_hbm, cand_hbm, buf, tbuf, obuf):
        wid = jax.lax.axis_index("s") * 2 + jax.lax.axis_index("c")
        b = wid // nsh
        sh = wid % nsh
        lanes8 = jax.lax.iota(jnp.int32, 16) * K_TOP
        neg = jnp.full((16,), -jnp.inf, dtype=jnp.float32)

        for g in range(G):
            for kk in range(K_TOP):
                tbuf[g * K_TOP + kk] = neg

        row0 = sh * rows

        def do_chunk(ci, _):
            pltpu.sync_copy(
                x_hbm.at[b + b_lo, pl.ds(row0 + ci * chunk, chunk), :], buf
            )

            for g in range(G):
                T = tuple(tbuf[g * K_TOP + kk] for kk in range(K_TOP))

                def do_rows(r, T, g=g):
                    base = r * _GRP
                    v = [
                        buf[base + i, g * 16:(g + 1) * 16]
                        for i in range(_GRP)
                    ]
                    m = v[0]
                    lvl = list(v)
                    while len(lvl) > 1:
                        lvl = [
                            jnp.maximum(lvl[2 * i], lvl[2 * i + 1])
                            for i in range(len(lvl) // 2)
                        ]
                    m = lvl[0]
                    cnt = plsc.all_reduce_population_count(m > T[K_TOP - 1])
                    pred = cnt[0] > 0

                    def slow(T):
                        for i in range(_GRP):
                            T = _insert(T, v[i])
                        return T

                    return jax.lax.cond(pred, slow, lambda T: T, T)

                T = jax.lax.fori_loop(0, chunk // _GRP, do_rows, T)
                for kk in range(K_TOP):
                    tbuf[g * K_TOP + kk] = T[kk]
            return 0

        jax.lax.fori_loop(0, nchunks, do_chunk, 0)

        for g in range(G):
            for kk in range(K_TOP):
                plsc.store_scatter(
                    obuf, [jnp.full((16,), g, jnp.int32), lanes8 + kk],
                    tbuf[g * K_TOP + kk],
                )
        pltpu.sync_copy(obuf, cand_hbm.at[b, pl.ds(sh * G, G), :])

    cand = k1(x)
    if nsh == 1:
        return cand.reshape(nb, C * K_TOP)

    @functools.partial(
        pl.kernel,
        out_type=jax.ShapeDtypeStruct((nb, G, 8 * 16), jnp.float32),
        mesh=mesh,
        scratch_types=[
            pltpu.VMEM((nsh * G, 8 * 16), jnp.float32),
            pltpu.VMEM((G, 8 * 16), jnp.float32),
        ],
        compiler_params=pltpu.CompilerParams(needs_layout_passes=False),
    )
    def k2(cand_hbm, out_hbm, cbuf, obuf):
        wid = jax.lax.axis_index("s") * 2 + jax.lax.axis_index("c")
        lanes8 = jax.lax.iota(jnp.int32, 16) * K_TOP
        neg = jnp.full((16,), -jnp.inf, dtype=jnp.float32)

        @pl.when(wid < nb)
        def _():
            pltpu.sync_copy(cand_hbm.at[wid], cbuf)

            def do_group(g, _):
                T = tuple([neg] * K_TOP)
                for sh in range(nsh):
                    for kk in range(K_TOP):
                        row = jnp.full((16,), sh * G + g, jnp.int32)
                        v = plsc.load_gather(cbuf, [row, lanes8 + kk])
                        T = _insert(T, v)
                for kk in range(K_TOP):
                    plsc.store_scatter(
                        obuf, [jnp.full((16,), g, jnp.int32), lanes8 + kk],
                        T[kk],
                    )
                return 0

            jax.lax.fori_loop(0, G, do_group, 0)
            pltpu.sync_copy(obuf, out_hbm.at[wid])

    return k2(cand).reshape(nb, C * K_TOP)


_SC_BATCHES = 8


def kernel(inputs):
    b_sc = _SC_BATCHES
    B = inputs.shape[0]
    out_tc = _kmax_tc(inputs, b_sc, B)
    out_sc = _kmax_sc(inputs, 0, b_sc)
    return jnp.concatenate([out_sc, out_tc], axis=0)


# speedup vs baseline: 4.3188x; 1.3547x over previous
import functools

import jax
import jax.numpy as jnp
from jax.experimental import pallas as pl
from jax.experimental.pallas import tpu as pltpu

K_TOP = 8


def _tc_body(x_ref, out_ref):
    S = x_ref.shape[1]
    C = x_ref.shape[2]
    nstep = S // 8
    neg = jnp.full((8, C), -jnp.inf, dtype=jnp.float32)

    def step(i, T):
        d = x_ref[0, pl.ds(i * 8, 8), :]
        out = []
        for k in range(K_TOP):
            t = T[k]
            out.append(jnp.maximum(t, d))
            if k < K_TOP - 1:
                d = jnp.minimum(t, d)
        return tuple(out)

    T = jax.lax.fori_loop(0, nstep, step, tuple([neg] * K_TOP), unroll=4)

    cand = jnp.concatenate(T, axis=0)
    ridx = jax.lax.broadcasted_iota(jnp.int32, (8 * K_TOP, C), 0)
    outs = []
    for _ in range(K_TOP):
        m = jnp.max(cand, axis=0)
        eq = cand == m[None, :]
        first = jnp.min(jnp.where(eq, ridx, 8 * K_TOP), axis=0)
        cand = jnp.where(eq & (ridx == first[None, :]), -jnp.inf, cand)
        outs.append(m)
    res = jnp.stack(outs, axis=0)
    out_ref[0] = jnp.transpose(res, (1, 0))


def _kmax_tc(x, b_lo=0, b_hi=None):
    B, S, C = x.shape
    if b_hi is None:
        b_hi = B
    nb = b_hi - b_lo
    out = pl.pallas_call(
        _tc_body,
        grid=(nb,),
        in_specs=[pl.BlockSpec((1, S, C), lambda b: (b + b_lo, 0, 0))],
        out_specs=pl.BlockSpec((1, C, K_TOP), lambda b: (b, 0, 0)),
        out_shape=jax.ShapeDtypeStruct((nb, C, K_TOP), jnp.float32),
    )(x)
    return out.reshape(nb, C * K_TOP)


_SORT8 = [
    (0, 1), (2, 3), (4, 5), (6, 7),
    (0, 2), (1, 3), (4, 6), (5, 7),
    (1, 2), (5, 6),
    (0, 4), (1, 5), (2, 6), (3, 7),
    (2, 4), (3, 5),
    (1, 2), (3, 4), (5, 6),
]

_BITONIC8 = [
    (0, 4), (1, 5), (2, 6), (3, 7),
    (0, 2), (1, 3), (4, 6), (5, 7),
    (0, 1), (2, 3), (4, 5), (6, 7),
]


def _insert(T, v):
    out = []
    d = v
    for kk in range(K_TOP):
        tk = T[kk]
        out.append(jnp.maximum(tk, d))
        if kk < K_TOP - 1:
            d = jnp.minimum(tk, d)
    return tuple(out)


def _kmax_sc(x, b_lo=0, b_hi=None, chunk=256):
    from jax.experimental.pallas import tpu_sc as plsc

    B, S, C = x.shape
    if b_hi is None:
        b_hi = B
    nb = b_hi - b_lo
    G = C // 16
    NW = 32
    assert NW % nb == 0
    nsh = NW // nb
    rows = S // nsh
    nchunks = rows // chunk
    assert rows % chunk == 0 and chunk % 8 == 0

    mesh = plsc.VectorSubcoreMesh(core_axis_name="c", subcore_axis_name="s")

    @functools.partial(
        pl.kernel,
        out_type=jax.ShapeDtypeStruct((nb, nsh * G, 8 * 16), jnp.float32),
        mesh=mesh,
        scratch_types=[
            pltpu.VMEM((chunk, C), jnp.float32),
            pltpu.VMEM((G * K_TOP, 16), jnp.float32),
            pltpu.VMEM((G, 8 * 16), jnp.float32),
        ],
        compiler_params=pltpu.CompilerParams(needs_layout_passes=False),
    )
    def k1(x_hbm, cand_hbm, buf, tbuf, obuf):
        wid = jax.lax.axis_index("s") * 2 + jax.lax.axis_index("c")
        b = wid // nsh
        sh = wid % nsh
        lanes8 = jax.lax.iota(jnp.int32, 16) * K_TOP
        neg = jnp.full((16,), -jnp.inf, dtype=jnp.float32)

        for g in range(G):
            for kk in range(K_TOP):
                tbuf[g * K_TOP + kk] = neg

        row0 = sh * rows

        def do_chunk(ci, _):
            pltpu.sync_copy(
                x_hbm.at[b + b_lo, pl.ds(row0 + ci * chunk, chunk), :], buf
            )

            for g in range(G):
                T = tuple(tbuf[g * K_TOP + kk] for kk in range(K_TOP))

                def do_rows(r, T, g=g):
                    base = r * 8
                    s = [
                        buf[base + i, g * 16:(g + 1) * 16]
                        for i in range(8)
                    ]
                    for (i, j) in _SORT8:
                        hi = jnp.maximum(s[i], s[j])
                        lo = jnp.minimum(s[i], s[j])
                        s[i], s[j] = hi, lo
                    c = [
                        jnp.maximum(s[i], T[K_TOP - 1 - i]) for i in range(8)
                    ]
                    for (i, j) in _BITONIC8:
                        hi = jnp.maximum(c[i], c[j])
                        lo = jnp.minimum(c[i], c[j])
                        c[i], c[j] = hi, lo
                    return tuple(c)

                T = jax.lax.fori_loop(0, chunk // 8, do_rows, T)
                for kk in range(K_TOP):
                    tbuf[g * K_TOP + kk] = T[kk]
            return 0

        jax.lax.fori_loop(0, nchunks, do_chunk, 0)

        for g in range(G):
            for kk in range(K_TOP):
                plsc.store_scatter(
                    obuf, [jnp.full((16,), g, jnp.int32), lanes8 + kk],
                    tbuf[g * K_TOP + kk],
                )
        pltpu.sync_copy(obuf, cand_hbm.at[b, pl.ds(sh * G, G), :])

    cand = k1(x)
    if nsh == 1:
        return cand.reshape(nb, C * K_TOP)

    @functools.partial(
        pl.kernel,
        out_type=jax.ShapeDtypeStruct((nb, G, 8 * 16), jnp.float32),
        mesh=mesh,
        scratch_types=[
            pltpu.VMEM((nsh * G, 8 * 16), jnp.float32),
            pltpu.VMEM((G, 8 * 16), jnp.float32),
        ],
        compiler_params=pltpu.CompilerParams(needs_layout_passes=False),
    )
    def k2(cand_hbm, out_hbm, cbuf, obuf):
        wid = jax.lax.axis_index("s") * 2 + jax.lax.axis_index("c")
        lanes8 = jax.lax.iota(jnp.int32, 16) * K_TOP
        neg = jnp.full((16,), -jnp.inf, dtype=jnp.float32)

        @pl.when(wid < nb)
        def _():
            pltpu.sync_copy(cand_hbm.at[wid], cbuf)

            def do_group(g, _):
                T = tuple([neg] * K_TOP)
                for sh in range(nsh):
                    for kk in range(K_TOP):
                        row = jnp.full((16,), sh * G + g, jnp.int32)
                        v = plsc.load_gather(cbuf, [row, lanes8 + kk])
                        T = _insert(T, v)
                for kk in range(K_TOP):
                    plsc.store_scatter(
                        obuf, [jnp.full((16,), g, jnp.int32), lanes8 + kk],
                        T[kk],
                    )
                return 0

            jax.lax.fori_loop(0, G, do_group, 0)
            pltpu.sync_copy(obuf, out_hbm.at[wid])

    return k2(cand).reshape(nb, C * K_TOP)


_SC_BATCHES = 8


def kernel(inputs):
    b_sc = _SC_BATCHES
    B = inputs.shape[0]
    out_tc = _kmax_tc(inputs, b_sc, B)
    out_sc = _kmax_sc(inputs, 0, b_sc)
    return jnp.concatenate([out_sc, out_tc], axis=0)


# speedup vs baseline: 5.4013x; 1.2507x over previous
import functools

import jax
import jax.numpy as jnp
from jax.experimental import pallas as pl
from jax.experimental.pallas import tpu as pltpu

K_TOP = 8


def _tc_body(x_ref, out_ref):
    S = x_ref.shape[1]
    C = x_ref.shape[2]
    nstep = S // 8
    neg = jnp.full((8, C), -jnp.inf, dtype=jnp.float32)

    def step(i, T):
        s = [x_ref[0, pl.ds(i * 64 + 8 * j, 8), :] for j in range(8)]
        for (a, b) in _SORT8:
            hi = jnp.maximum(s[a], s[b])
            lo = jnp.minimum(s[a], s[b])
            s[a], s[b] = hi, lo
        c = [jnp.maximum(s[j], T[K_TOP - 1 - j]) for j in range(8)]
        for (a, b) in _BITONIC8:
            hi = jnp.maximum(c[a], c[b])
            lo = jnp.minimum(c[a], c[b])
            c[a], c[b] = hi, lo
        return tuple(c)

    T = jax.lax.fori_loop(0, nstep // 8, step, tuple([neg] * K_TOP), unroll=2)

    cand = jnp.concatenate(T, axis=0)
    ridx = jax.lax.broadcasted_iota(jnp.int32, (8 * K_TOP, C), 0)
    outs = []
    for _ in range(K_TOP):
        m = jnp.max(cand, axis=0)
        eq = cand == m[None, :]
        first = jnp.min(jnp.where(eq, ridx, 8 * K_TOP), axis=0)
        cand = jnp.where(eq & (ridx == first[None, :]), -jnp.inf, cand)
        outs.append(m)
    res = jnp.stack(outs, axis=0)
    out_ref[0] = jnp.transpose(res, (1, 0))


def _kmax_tc(x, b_lo=0, b_hi=None):
    B, S, C = x.shape
    if b_hi is None:
        b_hi = B
    nb = b_hi - b_lo
    out = pl.pallas_call(
        _tc_body,
        grid=(nb,),
        in_specs=[pl.BlockSpec((1, S, C), lambda b: (b + b_lo, 0, 0))],
        out_specs=pl.BlockSpec((1, C, K_TOP), lambda b: (b, 0, 0)),
        out_shape=jax.ShapeDtypeStruct((nb, C, K_TOP), jnp.float32),
    )(x)
    return out.reshape(nb, C * K_TOP)


_SORT8 = [
    (0, 1), (2, 3), (4, 5), (6, 7),
    (0, 2), (1, 3), (4, 6), (5, 7),
    (1, 2), (5, 6),
    (0, 4), (1, 5), (2, 6), (3, 7),
    (2, 4), (3, 5),
    (1, 2), (3, 4), (5, 6),
]

_BITONIC8 = [
    (0, 4), (1, 5), (2, 6), (3, 7),
    (0, 2), (1, 3), (4, 6), (5, 7),
    (0, 1), (2, 3), (4, 5), (6, 7),
]


def _insert(T, v):
    out = []
    d = v
    for kk in range(K_TOP):
        tk = T[kk]
        out.append(jnp.maximum(tk, d))
        if kk < K_TOP - 1:
            d = jnp.minimum(tk, d)
    return tuple(out)


def _kmax_sc(x, b_lo=0, b_hi=None, chunk=256):
    from jax.experimental.pallas import tpu_sc as plsc

    B, S, C = x.shape
    if b_hi is None:
        b_hi = B
    nb = b_hi - b_lo
    G = C // 16
    NW = 32
    assert NW % nb == 0
    nsh = NW // nb
    rows = S // nsh
    nchunks = rows // chunk
    assert rows % chunk == 0 and chunk % 8 == 0

    mesh = plsc.VectorSubcoreMesh(core_axis_name="c", subcore_axis_name="s")

    @functools.partial(
        pl.kernel,
        out_type=jax.ShapeDtypeStruct((nb, nsh * G, 8 * 16), jnp.float32),
        mesh=mesh,
        scratch_types=[
            pltpu.VMEM((chunk, C), jnp.float32),
            pltpu.VMEM((G * K_TOP, 16), jnp.float32),
            pltpu.VMEM((G, 8 * 16), jnp.float32),
        ],
        compiler_params=pltpu.CompilerParams(needs_layout_passes=False),
    )
    def k1(x_hbm, cand_hbm, buf, tbuf, obuf):
        wid = jax.lax.axis_index("s") * 2 + jax.lax.axis_index("c")
        b = wid // nsh
        sh = wid % nsh
        lanes8 = jax.lax.iota(jnp.int32, 16) * K_TOP
        neg = jnp.full((16,), -jnp.inf, dtype=jnp.float32)

        for g in range(G):
            for kk in range(K_TOP):
                tbuf[g * K_TOP + kk] = neg

        row0 = sh * rows

        def do_chunk(ci, _):
            pltpu.sync_copy(
                x_hbm.at[b + b_lo, pl.ds(row0 + ci * chunk, chunk), :], buf
            )

            for g in range(G):
                T = tuple(tbuf[g * K_TOP + kk] for kk in range(K_TOP))

                def do_rows(r, T, g=g):
                    base = r * 8
                    s = [
                        buf[base + i, g * 16:(g + 1) * 16]
                        for i in range(8)
                    ]
                    for (i, j) in _SORT8:
                        hi = jnp.maximum(s[i], s[j])
                        lo = jnp.minimum(s[i], s[j])
                        s[i], s[j] = hi, lo
                    c = [
                        jnp.maximum(s[i], T[K_TOP - 1 - i]) for i in range(8)
                    ]
                    for (i, j) in _BITONIC8:
                        hi = jnp.maximum(c[i], c[j])
                        lo = jnp.minimum(c[i], c[j])
                        c[i], c[j] = hi, lo
                    return tuple(c)

                T = jax.lax.fori_loop(0, chunk // 8, do_rows, T)
                for kk in range(K_TOP):
                    tbuf[g * K_TOP + kk] = T[kk]
            return 0

        jax.lax.fori_loop(0, nchunks, do_chunk, 0)

        for g in range(G):
            for kk in range(K_TOP):
                plsc.store_scatter(
                    obuf, [jnp.full((16,), g, jnp.int32), lanes8 + kk],
                    tbuf[g * K_TOP + kk],
                )
        pltpu.sync_copy(obuf, cand_hbm.at[b, pl.ds(sh * G, G), :])

    cand = k1(x)
    if nsh == 1:
        return cand.reshape(nb, C * K_TOP)

    @functools.partial(
        pl.kernel,
        out_type=jax.ShapeDtypeStruct((nb, G, 8 * 16), jnp.float32),
        mesh=mesh,
        scratch_types=[
            pltpu.VMEM((nsh * G, 8 * 16), jnp.float32),
            pltpu.VMEM((G, 8 * 16), jnp.float32),
        ],
        compiler_params=pltpu.CompilerParams(needs_layout_passes=False),
    )
    def k2(cand_hbm, out_hbm, cbuf, obuf):
        wid = jax.lax.axis_index("s") * 2 + jax.lax.axis_index("c")
        lanes8 = jax.lax.iota(jnp.int32, 16) * K_TOP
        neg = jnp.full((16,), -jnp.inf, dtype=jnp.float32)

        @pl.when(wid < nb)
        def _():
            pltpu.sync_copy(cand_hbm.at[wid], cbuf)

            def do_group(g, _):
                T = tuple([neg] * K_TOP)
                for sh in range(nsh):
                    for kk in range(K_TOP):
                        row = jnp.full((16,), sh * G + g, jnp.int32)
                        v = plsc.load_gather(cbuf, [row, lanes8 + kk])
                        T = _insert(T, v)
                for kk in range(K_TOP):
                    plsc.store_scatter(
                        obuf, [jnp.full((16,), g, jnp.int32), lanes8 + kk],
                        T[kk],
                    )
                return 0

            jax.lax.fori_loop(0, G, do_group, 0)
            pltpu.sync_copy(obuf, out_hbm.at[wid])

    return k2(cand).reshape(nb, C * K_TOP)


_SC_BATCHES = 8


def kernel(inputs):
    b_sc = _SC_BATCHES
    B = inputs.shape[0]
    out_tc = _kmax_tc(inputs, b_sc, B)
    out_sc = _kmax_sc(inputs, 0, b_sc)
    return jnp.concatenate([out_sc, out_tc], axis=0)


# speedup vs baseline: 5.9730x; 1.1058x over previous
import functools

import jax
import jax.numpy as jnp
from jax.experimental import pallas as pl
from jax.experimental.pallas import tpu as pltpu

K_TOP = 8


def _tc_body(x_ref, out_ref):
    S = x_ref.shape[1]
    C = x_ref.shape[2]
    nstep = S // 8
    neg = jnp.full((8, C), -jnp.inf, dtype=jnp.float32)

    def step(i, T):
        s = [x_ref[0, pl.ds(i * 64 + 8 * j, 8), :] for j in range(8)]
        for (a, b) in _SORT8:
            hi = jnp.maximum(s[a], s[b])
            lo = jnp.minimum(s[a], s[b])
            s[a], s[b] = hi, lo
        c = [jnp.maximum(s[j], T[K_TOP - 1 - j]) for j in range(8)]
        for (a, b) in _BITONIC8:
            hi = jnp.maximum(c[a], c[b])
            lo = jnp.minimum(c[a], c[b])
            c[a], c[b] = hi, lo
        return tuple(c)

    T = jax.lax.fori_loop(0, nstep // 8, step, tuple([neg] * K_TOP), unroll=2)

    cand = jnp.concatenate(T, axis=0)
    ridx = jax.lax.broadcasted_iota(jnp.int32, (8 * K_TOP, C), 0)
    outs = []
    for _ in range(K_TOP):
        m = jnp.max(cand, axis=0)
        eq = cand == m[None, :]
        first = jnp.min(jnp.where(eq, ridx, 8 * K_TOP), axis=0)
        cand = jnp.where(eq & (ridx == first[None, :]), -jnp.inf, cand)
        outs.append(m)
    res = jnp.stack(outs, axis=0)
    out_ref[0] = jnp.transpose(res, (1, 0))


def _kmax_tc(x, b_lo=0, b_hi=None):
    B, S, C = x.shape
    if b_hi is None:
        b_hi = B
    nb = b_hi - b_lo
    out = pl.pallas_call(
        _tc_body,
        grid=(nb,),
        in_specs=[pl.BlockSpec((1, S, C), lambda b: (b + b_lo, 0, 0))],
        out_specs=pl.BlockSpec((1, C, K_TOP), lambda b: (b, 0, 0)),
        out_shape=jax.ShapeDtypeStruct((nb, C, K_TOP), jnp.float32),
    )(x)
    return out.reshape(nb, C * K_TOP)


_SORT8 = [
    (0, 1), (2, 3), (4, 5), (6, 7),
    (0, 2), (1, 3), (4, 6), (5, 7),
    (1, 2), (5, 6),
    (0, 4), (1, 5), (2, 6), (3, 7),
    (2, 4), (3, 5),
    (1, 2), (3, 4), (5, 6),
]

_BITONIC8 = [
    (0, 4), (1, 5), (2, 6), (3, 7),
    (0, 2), (1, 3), (4, 6), (5, 7),
    (0, 1), (2, 3), (4, 5), (6, 7),
]


def _insert(T, v):
    out = []
    d = v
    for kk in range(K_TOP):
        tk = T[kk]
        out.append(jnp.maximum(tk, d))
        if kk < K_TOP - 1:
            d = jnp.minimum(tk, d)
    return tuple(out)


def _kmax_sc(x, b_lo=0, b_hi=None, chunk=256):
    from jax.experimental.pallas import tpu_sc as plsc

    B, S, C = x.shape
    if b_hi is None:
        b_hi = B
    nb = b_hi - b_lo
    G = C // 16
    NW = 32
    assert NW % nb == 0
    nsh = NW // nb
    rows = S // nsh
    nchunks = rows // chunk
    assert rows % chunk == 0 and chunk % 8 == 0

    mesh = plsc.VectorSubcoreMesh(core_axis_name="c", subcore_axis_name="s")

    @functools.partial(
        pl.kernel,
        out_type=jax.ShapeDtypeStruct((nb, nsh * G, 8 * 16), jnp.float32),
        mesh=mesh,
        scratch_types=[
            pltpu.VMEM((chunk, C), jnp.float32),
            pltpu.VMEM((G * K_TOP, 16), jnp.float32),
            pltpu.VMEM((G, 8 * 16), jnp.float32),
        ],
        compiler_params=pltpu.CompilerParams(needs_layout_passes=False),
    )
    def k1(x_hbm, cand_hbm, buf, tbuf, obuf):
        wid = jax.lax.axis_index("s") * 2 + jax.lax.axis_index("c")
        b = wid // nsh
        sh = wid % nsh
        lanes8 = jax.lax.iota(jnp.int32, 16) * K_TOP
        neg = jnp.full((16,), -jnp.inf, dtype=jnp.float32)

        for g in range(G):
            for kk in range(K_TOP):
                tbuf[g * K_TOP + kk] = neg

        row0 = sh * rows

        def do_chunk(ci, _):
            pltpu.sync_copy(
                x_hbm.at[b + b_lo, pl.ds(row0 + ci * chunk, chunk), :], buf
            )

            for g in range(G):
                T = tuple(tbuf[g * K_TOP + kk] for kk in range(K_TOP))

                def do_rows(r, T, g=g):
                    base = r * 8
                    s = [
                        buf[base + i, g * 16:(g + 1) * 16]
                        for i in range(8)
                    ]
                    for (i, j) in _SORT8:
                        hi = jnp.maximum(s[i], s[j])
                        lo = jnp.minimum(s[i], s[j])
                        s[i], s[j] = hi, lo
                    c = [
                        jnp.maximum(s[i], T[K_TOP - 1 - i]) for i in range(8)
                    ]
                    for (i, j) in _BITONIC8:
                        hi = jnp.maximum(c[i], c[j])
                        lo = jnp.minimum(c[i], c[j])
                        c[i], c[j] = hi, lo
                    return tuple(c)

                T = jax.lax.fori_loop(0, chunk // 8, do_rows, T)
                for kk in range(K_TOP):
                    tbuf[g * K_TOP + kk] = T[kk]
            return 0

        jax.lax.fori_loop(0, nchunks, do_chunk, 0)

        for g in range(G):
            for kk in range(K_TOP):
                plsc.store_scatter(
                    obuf, [jnp.full((16,), g, jnp.int32), lanes8 + kk],
                    tbuf[g * K_TOP + kk],
                )
        pltpu.sync_copy(obuf, cand_hbm.at[b, pl.ds(sh * G, G), :])

    cand = k1(x)
    if nsh == 1:
        return cand.reshape(nb, C * K_TOP)

    @functools.partial(
        pl.kernel,
        out_type=jax.ShapeDtypeStruct((nb, G, 8 * 16), jnp.float32),
        mesh=mesh,
        scratch_types=[
            pltpu.VMEM((nsh * G, 8 * 16), jnp.float32),
            pltpu.VMEM((G, 8 * 16), jnp.float32),
        ],
        compiler_params=pltpu.CompilerParams(needs_layout_passes=False),
    )
    def k2(cand_hbm, out_hbm, cbuf, obuf):
        wid = jax.lax.axis_index("s") * 2 + jax.lax.axis_index("c")
        lanes8 = jax.lax.iota(jnp.int32, 16) * K_TOP
        neg = jnp.full((16,), -jnp.inf, dtype=jnp.float32)

        @pl.when(wid < nb)
        def _():
            pltpu.sync_copy(cand_hbm.at[wid], cbuf)

            def do_group(g, _):
                T = tuple([neg] * K_TOP)
                for sh in range(nsh):
                    for kk in range(K_TOP):
                        row = jnp.full((16,), sh * G + g, jnp.int32)
                        v = plsc.load_gather(cbuf, [row, lanes8 + kk])
                        T = _insert(T, v)
                for kk in range(K_TOP):
                    plsc.store_scatter(
                        obuf, [jnp.full((16,), g, jnp.int32), lanes8 + kk],
                        T[kk],
                    )
                return 0

            jax.lax.fori_loop(0, G, do_group, 0)
            pltpu.sync_copy(obuf, out_hbm.at[wid])

    return k2(cand).reshape(nb, C * K_TOP)


_SC_BATCHES = 4


def kernel(inputs):
    b_sc = _SC_BATCHES
    B = inputs.shape[0]
    out_tc = _kmax_tc(inputs, b_sc, B)
    out_sc = _kmax_sc(inputs, 0, b_sc)
    return jnp.concatenate([out_sc, out_tc], axis=0)


# speedup vs baseline: 6.1249x; 1.0254x over previous
import functools

import jax
import jax.numpy as jnp
from jax.experimental import pallas as pl
from jax.experimental.pallas import tpu as pltpu

K_TOP = 8


def _tc_body(x_ref, out_ref):
    S = x_ref.shape[1]
    C = x_ref.shape[2]
    nstep = S // 8
    neg = jnp.full((8, C), -jnp.inf, dtype=jnp.float32)

    def step(i, T):
        s = [x_ref[0, pl.ds(i * 64 + 8 * j, 8), :] for j in range(8)]
        for (a, b) in _SORT8:
            hi = jnp.maximum(s[a], s[b])
            lo = jnp.minimum(s[a], s[b])
            s[a], s[b] = hi, lo
        c = [jnp.maximum(s[j], T[K_TOP - 1 - j]) for j in range(8)]
        for (a, b) in _BITONIC8:
            hi = jnp.maximum(c[a], c[b])
            lo = jnp.minimum(c[a], c[b])
            c[a], c[b] = hi, lo
        return tuple(c)

    T = jax.lax.fori_loop(0, nstep // 8, step, tuple([neg] * K_TOP), unroll=4)

    cand = jnp.concatenate(T, axis=0)
    ridx = jax.lax.broadcasted_iota(jnp.int32, (8 * K_TOP, C), 0)
    outs = []
    for _ in range(K_TOP):
        m = jnp.max(cand, axis=0)
        eq = cand == m[None, :]
        first = jnp.min(jnp.where(eq, ridx, 8 * K_TOP), axis=0)
        cand = jnp.where(eq & (ridx == first[None, :]), -jnp.inf, cand)
        outs.append(m)
    res = jnp.stack(outs, axis=0)
    out_ref[0] = jnp.transpose(res, (1, 0))


def _kmax_tc(x, b_lo=0, b_hi=None):
    B, S, C = x.shape
    if b_hi is None:
        b_hi = B
    nb = b_hi - b_lo
    out = pl.pallas_call(
        _tc_body,
        grid=(nb,),
        in_specs=[pl.BlockSpec((1, S, C), lambda b: (b + b_lo, 0, 0))],
        out_specs=pl.BlockSpec((1, C, K_TOP), lambda b: (b, 0, 0)),
        out_shape=jax.ShapeDtypeStruct((nb, C, K_TOP), jnp.float32),
    )(x)
    return out.reshape(nb, C * K_TOP)


_SORT8 = [
    (0, 1), (2, 3), (4, 5), (6, 7),
    (0, 2), (1, 3), (4, 6), (5, 7),
    (1, 2), (5, 6),
    (0, 4), (1, 5), (2, 6), (3, 7),
    (2, 4), (3, 5),
    (1, 2), (3, 4), (5, 6),
]

_BITONIC8 = [
    (0, 4), (1, 5), (2, 6), (3, 7),
    (0, 2), (1, 3), (4, 6), (5, 7),
    (0, 1), (2, 3), (4, 5), (6, 7),
]


def _insert(T, v):
    out = []
    d = v
    for kk in range(K_TOP):
        tk = T[kk]
        out.append(jnp.maximum(tk, d))
        if kk < K_TOP - 1:
            d = jnp.minimum(tk, d)
    return tuple(out)


def _kmax_sc(x, b_lo=0, b_hi=None, chunk=256):
    from jax.experimental.pallas import tpu_sc as plsc

    B, S, C = x.shape
    if b_hi is None:
        b_hi = B
    nb = b_hi - b_lo
    G = C // 16
    NW = 32
    assert NW % nb == 0
    nsh = NW // nb
    rows = S // nsh
    nchunks = rows // chunk
    assert rows % chunk == 0 and chunk % 8 == 0

    mesh = plsc.VectorSubcoreMesh(core_axis_name="c", subcore_axis_name="s")

    @functools.partial(
        pl.kernel,
        out_type=jax.ShapeDtypeStruct((nb, nsh * G, 8 * 16), jnp.float32),
        mesh=mesh,
        scratch_types=[
            pltpu.VMEM((chunk, C), jnp.float32),
            pltpu.VMEM((G * K_TOP, 16), jnp.float32),
            pltpu.VMEM((G, 8 * 16), jnp.float32),
        ],
        compiler_params=pltpu.CompilerParams(needs_layout_passes=False),
    )
    def k1(x_hbm, cand_hbm, buf, tbuf, obuf):
        wid = jax.lax.axis_index("s") * 2 + jax.lax.axis_index("c")
        b = wid // nsh
        sh = wid % nsh
        lanes8 = jax.lax.iota(jnp.int32, 16) * K_TOP
        neg = jnp.full((16,), -jnp.inf, dtype=jnp.float32)

        for g in range(G):
            for kk in range(K_TOP):
                tbuf[g * K_TOP + kk] = neg

        row0 = sh * rows

        def do_chunk(ci, _):
            pltpu.sync_copy(
                x_hbm.at[b + b_lo, pl.ds(row0 + ci * chunk, chunk), :], buf
            )

            for g in range(G):
                T = tuple(tbuf[g * K_TOP + kk] for kk in range(K_TOP))

                def do_rows(r, T, g=g):
                    base = r * 8
                    s = [
                        buf[base + i, g * 16:(g + 1) * 16]
                        for i in range(8)
                    ]
                    for (i, j) in _SORT8:
                        hi = jnp.maximum(s[i], s[j])
                        lo = jnp.minimum(s[i], s[j])
                        s[i], s[j] = hi, lo
                    c = [
                        jnp.maximum(s[i], T[K_TOP - 1 - i]) for i in range(8)
                    ]
                    for (i, j) in _BITONIC8:
                        hi = jnp.maximum(c[i], c[j])
                        lo = jnp.minimum(c[i], c[j])
                        c[i], c[j] = hi, lo
                    return tuple(c)

                T = jax.lax.fori_loop(0, chunk // 8, do_rows, T)
                for kk in range(K_TOP):
                    tbuf[g * K_TOP + kk] = T[kk]
            return 0

        jax.lax.fori_loop(0, nchunks, do_chunk, 0)

        for g in range(G):
            for kk in range(K_TOP):
                plsc.store_scatter(
                    obuf, [jnp.full((16,), g, jnp.int32), lanes8 + kk],
                    tbuf[g * K_TOP + kk],
                )
        pltpu.sync_copy(obuf, cand_hbm.at[b, pl.ds(sh * G, G), :])

    cand = k1(x)
    if nsh == 1:
        return cand.reshape(nb, C * K_TOP)

    @functools.partial(
        pl.kernel,
        out_type=jax.ShapeDtypeStruct((nb, G, 8 * 16), jnp.float32),
        mesh=mesh,
        scratch_types=[
            pltpu.VMEM((nsh * G, 8 * 16), jnp.float32),
            pltpu.VMEM((G, 8 * 16), jnp.float32),
        ],
        compiler_params=pltpu.CompilerParams(needs_layout_passes=False),
    )
    def k2(cand_hbm, out_hbm, cbuf, obuf):
        wid = jax.lax.axis_index("s") * 2 + jax.lax.axis_index("c")
        lanes8 = jax.lax.iota(jnp.int32, 16) * K_TOP
        neg = jnp.full((16,), -jnp.inf, dtype=jnp.float32)

        @pl.when(wid < nb)
        def _():
            pltpu.sync_copy(cand_hbm.at[wid], cbuf)

            def do_group(g, _):
                T = tuple([neg] * K_TOP)
                for sh in range(nsh):
                    for kk in range(K_TOP):
                        row = jnp.full((16,), sh * G + g, jnp.int32)
                        v = plsc.load_gather(cbuf, [row, lanes8 + kk])
                        T = _insert(T, v)
                for kk in range(K_TOP):
                    plsc.store_scatter(
                        obuf, [jnp.full((16,), g, jnp.int32), lanes8 + kk],
                        T[kk],
                    )
                return 0

            jax.lax.fori_loop(0, G, do_group, 0)
            pltpu.sync_copy(obuf, out_hbm.at[wid])

    return k2(cand).reshape(nb, C * K_TOP)


_SC_BATCHES = 4


def kernel(inputs):
    b_sc = _SC_BATCHES
    B = inputs.shape[0]
    out_tc = _kmax_tc(inputs, b_sc, B)
    out_sc = _kmax_sc(inputs, 0, b_sc)
    return jnp.concatenate([out_sc, out_tc], axis=0)
